# baseline (speedup 1.0000x reference)
import jax
import jax.numpy as jnp
from jax.experimental import pallas as pl

_B, _T = 4, 8192
_F = 4
_Q = _T // _F
_DM, _KLAT, _DMODEL = 64, 16, 128
_DIN, _DA = 64, 32
_KSEL, _KEFF = 8, 16
_NEG = -1e30


def _body(x4_ref, fb_ref, wu_ref, bur_ref, w4_ref, buc4_ref, a4_ref, bac4_ref,
          wmat_ref, m0r_ref, wsc_ref, bs_ref, pos_ref, wz_ref, bz_ref, wr_ref,
          br_ref, wh_ref, bh_ref, mu_ref, sig_ref, wl_ref, bl_ref, wp_ref,
          bp_ref, y_ref, tok_ref, mem_ref):
    f32 = jnp.float32
    i32 = jnp.int32
    m0r = m0r_ref[...]
    wsc = wsc_ref[...]
    mwa_c = jnp.sum(wmat_ref[...] * m0r, axis=1, keepdims=True)
    mwa4 = jnp.concatenate([mwa_c] * _F, axis=0)
    ig = (_F * jax.lax.broadcasted_iota(i32, (_F, _Q), 1)
          + jax.lax.broadcasted_iota(i32, (_F, _Q), 0))
    iota_l = jax.lax.broadcasted_iota(i32, (1, _T), 1)
    colio = jax.lax.broadcasted_iota(i32, (1, _Q), 1)
    ones_row = jnp.ones((1, _Q), f32)

    dn_t = (((0,), (1,)), ((), ()))
    dn_tt = (((0,), (0,)), ((), ()))

    for b in range(_B):
        _one_batch(b, x4_ref, fb_ref, wu_ref, bur_ref, w4_ref, buc4_ref,
                   a4_ref, bac4_ref, wsc, bs_ref, pos_ref, m0r, mwa4, ig,
                   iota_l, colio, ones_row, dn_t, dn_tt, wz_ref, bz_ref,
                   wr_ref, br_ref, wh_ref, bh_ref, mu_ref, sig_ref, wl_ref,
                   bl_ref, wp_ref, bp_ref, y_ref, tok_ref, mem_ref)


def _one_batch(b, x4_ref, fb_ref, wu_ref, bur_ref, w4_ref, buc4_ref,
               a4_ref, bac4_ref, wsc, bs_ref, pos_ref, m0r, mwa4, ig,
               iota_l, colio, ones_row, dn_t, dn_tt, wz_ref, bz_ref,
               wr_ref, br_ref, wh_ref, bh_ref, mu_ref, sig_ref, wl_ref,
               bl_ref, wp_ref, bp_ref, y_ref, tok_ref, mem_ref):
    f32 = jnp.float32
    i32 = jnp.int32
    xb = x4_ref[b]
    ut = jnp.tanh(
        jax.lax.dot_general(w4_ref[...], xb, dn_t, preferred_element_type=f32)
        + buc4_ref[...])
    at = jnp.tanh(
        jax.lax.dot_general(a4_ref[...], ut, dn_tt, preferred_element_type=f32)
        + mwa4 + bac4_ref[...])
    shs = [jnp.sum(at[_DA * h:_DA * (h + 1), :] * wsc, axis=0, keepdims=True)
           for h in range(_F)]
    s = jnp.concatenate(shs, axis=0) + bs_ref[...] + pos_ref[...]
    maxs = jnp.max(s, keepdims=True)
    sumexp = jnp.sum(jnp.exp(s - maxs), keepdims=True)

    ms = s
    pidxs, pjs = [], []
    for _ in range(_KSEL):
        v = jnp.max(ms, keepdims=True)
        pidx = jnp.min(jnp.where(ms == v, ig, _T), keepdims=True)
        pjs.append(jnp.exp(v - maxs) / sumexp)
        pidxs.append(pidx)
        ms = jnp.where(jnp.abs(ig - pidx) <= 1, _NEG, ms)

    p8r = jnp.concatenate(pjs, axis=1)
    i8r = jnp.concatenate(pidxs, axis=1)
    p8c = jnp.concatenate(pjs, axis=0)
    i8c = jnp.concatenate(pidxs, axis=0)

    y_row = jnp.zeros((1, _T), f32)
    for pidx, pj in zip(pidxs, pjs):
        y_row = y_row + pj * (iota_l == pidx).astype(f32)
    y_ref[b] = y_row
    sump = jnp.sum(p8r, keepdims=True)
    denom = sump + 1e-8

    before = (p8c > p8r) | ((p8c == p8r) & (i8c < i8r))
    rank = jnp.sum(before.astype(i32), axis=0, keepdims=True)
    k8c = jax.lax.broadcasted_iota(i32, (_KSEL, 1), 0)
    mrank = (rank == k8c).astype(f32)
    svals = jnp.sum(mrank * p8r, axis=1, keepdims=True)
    sidx = jnp.sum(mrank.astype(i32) * i8r, axis=1, keepdims=True)
    t16r = jax.lax.broadcasted_iota(i32, (1, 2 * _KSEL), 1)
    picked = jnp.zeros((1, 2 * _KSEL), jnp.bool_)
    for pidx in pidxs:
        picked = picked | (t16r == pidx)
    free = ~picked
    t16c = jax.lax.broadcasted_iota(i32, (2 * _KSEL, 1), 0)
    free_c = jnp.sum((t16c == t16r).astype(i32)
                     * free.astype(i32), axis=1, keepdims=True)
    bc = jnp.sum(jnp.where((t16c < t16r) & (free_c > 0), 1, 0),
                 axis=0, keepdims=True)
    m2 = ((bc == k8c) & free).astype(i32)
    zidx = jnp.sum(m2 * t16r, axis=1, keepdims=True)
    tii = jnp.concatenate([sidx, zidx], axis=0)
    tv = jnp.concatenate([svals, jnp.zeros((_KSEL, 1), f32)], axis=0)

    qi = tii // _F
    hsel = tii - _F * qi
    onehot = (qi == colio).astype(f32)
    xg256 = jnp.dot(onehot, xb, preferred_element_type=f32)
    xg = jnp.zeros((_KEFF, _DIN), f32)
    for h in range(_F):
        xg = xg + ((hsel == h).astype(f32)
                   * xg256[:, _DIN * h:_DIN * (h + 1)])
    xs = jnp.dot(ones_row, xb, preferred_element_type=f32)
    xmean = sum(xs[:, _DIN * h:_DIN * (h + 1)] for h in range(_F)) * (1.0 / _T)

    i8f = i8r.astype(f32)
    cn = jnp.sum(p8r * (i8r <= tii).astype(f32), axis=1,
                 keepdims=True) / denom
    mean_cn = jnp.sum(p8r * (_T - i8f), keepdims=True) / (denom * _T)

    posn = tii.astype(f32) * (1.0 / _T)
    dvec = jnp.concatenate([xg, tv, posn, cn], axis=1)
    mp = jnp.full((1, 1), (_T - 1) / (2.0 * _T), f32)
    dmean = jnp.concatenate(
        [xmean, sump * (1.0 / _T), mp, mean_cn], axis=1)
    c = dvec - dmean
    c = c / (jnp.sqrt(jnp.sum(c * c, axis=1, keepdims=True)) + 1e-6)
    zz = (c - mu_ref[...]) / sig_ref[...]
    lif = jnp.tanh(jnp.dot(zz, wl_ref[...], preferred_element_type=f32)
                   + bl_ref[...])
    lif = lif / (jnp.sqrt(jnp.sum(lif * lif, axis=1, keepdims=True)) + 1e-6)
    tok_ref[b] = (jnp.dot(lif, wp_ref[...], preferred_element_type=f32)
                  + bp_ref[...])

    u8 = jnp.tanh(jnp.dot(xg[0:_KSEL, :], wu_ref[...],
                          preferred_element_type=f32) + bur_ref[...])
    w8 = tv[0:_KSEL, :] / denom
    ctx = jnp.sum(w8 * u8, axis=0, keepdims=True)
    inp = jnp.concatenate([ctx, fb_ref[b]], axis=1)
    xh = jnp.concatenate([inp, m0r], axis=1)
    zg = jax.nn.sigmoid(jnp.dot(xh, wz_ref[...], preferred_element_type=f32)
                        + bz_ref[...])
    rg = jax.nn.sigmoid(jnp.dot(xh, wr_ref[...], preferred_element_type=f32)
                        + br_ref[...])
    xrh = jnp.concatenate([inp, rg * m0r], axis=1)
    hh = jnp.tanh(jnp.dot(xrh, wh_ref[...], preferred_element_type=f32)
                  + bh_ref[...])
    m1 = (1.0 - zg) * m0r + zg * hh
    mem_ref[b] = jnp.concatenate([m0r, m1], axis=0)


def kernel(x, feedback, params):
    p = params
    B, T, _ = x.shape
    f32 = jnp.float32
    x4 = x.reshape(B, _Q, _F * _DIN)
    wu = p['W_u']
    wa = p['W_a']
    w4 = jnp.zeros((_F * _DIN, _F * _DIN), f32)
    a4 = jnp.zeros((_F * _DIN, _F * _DA), f32)
    for h in range(_F):
        w4 = w4.at[_DIN * h:_DIN * (h + 1), _DIN * h:_DIN * (h + 1)].set(wu)
        a4 = a4.at[_DIN * h:_DIN * (h + 1), _DA * h:_DA * (h + 1)].set(wa)
    buc4 = jnp.tile(p['b_u'].reshape(-1, 1), (_F, 1))
    bac4 = jnp.tile(p['b_a'].reshape(-1, 1), (_F, 1))
    pos_f = p['pos_bias'][:T].reshape(_Q, _F).T

    all_y, tokens, mem = pl.pallas_call(
        _body,
        out_shape=(
            jax.ShapeDtypeStruct((B, 1, T), f32),
            jax.ShapeDtypeStruct((B, _KEFF, _DMODEL), f32),
            jax.ShapeDtypeStruct((B, 2, _DM), f32),
        ),
    )(
        x4, feedback.reshape(B, 1, 1),
        wu, p['b_u'].reshape(1, -1),
        w4, buc4, a4, bac4,
        p['W_ma'].T, p['m0'].reshape(1, -1),
        p['w_s'].reshape(-1, 1), p['b_s'].reshape(1, 1), pos_f,
        p['W_z'], p['b_z'].reshape(1, -1),
        p['W_r'], p['b_r'].reshape(1, -1),
        p['W_h'], p['b_h'].reshape(1, -1),
        p['mu'].reshape(1, -1), p['sigma'].reshape(1, -1),
        p['W_lift'], p['b_lift'].reshape(1, -1),
        p['W_proj'], p['b_proj'].reshape(1, -1),
    )
    y_star = all_y[:, 0, :]
    return tokens, y_star, all_y, mem


# speedup vs baseline: 1.3862x; 1.3862x over previous
import jax
import jax.numpy as jnp
from jax.experimental import pallas as pl

_B, _T = 4, 8192
_R = 8
_Q = _T // _R
_DM, _KLAT, _DMODEL = 64, 16, 128
_DIN, _DA = 64, 32
_KSEL, _KEFF = 8, 16
_NEG = -1e30


def _body(x_ref, fb_ref, wu_ref, bur_ref, buc_ref, wa_ref, bac_ref, wmat_ref,
          m0r_ref, wsc_ref, bs_ref, pos_ref, wz_ref, bz_ref, wr_ref, br_ref,
          wh_ref, bh_ref, mu_ref, sig_ref, wl_ref, bl_ref, wp_ref, bp_ref,
          y_ref, tok_ref, mem_ref):
    f32 = jnp.float32
    i32 = jnp.int32
    wu = wu_ref[...]
    m0r = m0r_ref[...]
    wsc = wsc_ref[...]
    mwa_c = jnp.sum(wmat_ref[...] * m0r, axis=1, keepdims=True)
    ig = (_Q * jax.lax.broadcasted_iota(i32, (_R, _Q), 0)
          + jax.lax.broadcasted_iota(i32, (_R, _Q), 1))
    iota_l = jax.lax.broadcasted_iota(i32, (1, _T), 1)

    dn_t = (((0,), (1,)), ((), ()))
    dn_tt = (((0,), (0,)), ((), ()))

    xb = x_ref[0]
    ut = jnp.tanh(
        jax.lax.dot_general(wu, xb, dn_t, preferred_element_type=f32)
        + buc_ref[...])
    at = jnp.tanh(
        jax.lax.dot_general(wa_ref[...], ut, dn_tt, preferred_element_type=f32)
        + mwa_c + bac_ref[...])
    s_row = (jnp.sum(at * wsc, axis=0, keepdims=True)
             + bs_ref[...] + pos_ref[...])
    s8 = jnp.concatenate(
        [s_row[:, _Q * i:_Q * (i + 1)] for i in range(_R)], axis=0)
    maxs = jnp.max(s8, keepdims=True)
    p_full = jnp.exp(s8 - maxs)
    sumexp = jnp.sum(p_full, keepdims=True)

    ms = s8
    selmask = jnp.zeros((_R, _Q), jnp.bool_)
    pidxs, pjs = [], []
    for _ in range(_KSEL):
        v = jnp.max(ms, keepdims=True)
        pidx = jnp.min(jnp.where(ms == v, ig, _T), keepdims=True)
        pjs.append(jnp.exp(v - maxs) / sumexp)
        pidxs.append(pidx)
        selmask = selmask | (ig == pidx)
        ms = jnp.where(jnp.abs(ig - pidx) <= 1, _NEG, ms)

    p8r = jnp.concatenate(pjs, axis=1)
    i8r = jnp.concatenate(pidxs, axis=1)
    p8c = jnp.concatenate(pjs, axis=0)
    i8c = jnp.concatenate(pidxs, axis=0)

    y8 = jnp.where(selmask, p_full / sumexp, 0.0)
    y_ref[0] = jnp.concatenate(
        [y8[i:i + 1, :] for i in range(_R)], axis=1)
    sump = jnp.sum(p8r, keepdims=True)
    denom = sump + 1e-8

    before = (p8c > p8r) | ((p8c == p8r) & (i8c < i8r))
    rank = jnp.sum(before.astype(i32), axis=0, keepdims=True)
    k8c = jax.lax.broadcasted_iota(i32, (_KSEL, 1), 0)
    mrank = (rank == k8c).astype(f32)
    svals = jnp.sum(mrank * p8r, axis=1, keepdims=True)
    sidx = jnp.sum(mrank.astype(i32) * i8r, axis=1, keepdims=True)
    t16r = jax.lax.broadcasted_iota(i32, (1, 2 * _KSEL), 1)
    picked = jnp.zeros((1, 2 * _KSEL), jnp.bool_)
    for pidx in pidxs:
        picked = picked | (t16r == pidx)
    free = ~picked
    t16c = jax.lax.broadcasted_iota(i32, (2 * _KSEL, 1), 0)
    free_c = jnp.sum((t16c == t16r).astype(i32)
                     * free.astype(i32), axis=1, keepdims=True)
    bc = jnp.sum(jnp.where((t16c < t16r) & (free_c > 0), 1, 0),
                 axis=0, keepdims=True)
    m2 = ((bc == k8c) & free).astype(i32)
    zidx = jnp.sum(m2 * t16r, axis=1, keepdims=True)
    tii = jnp.concatenate([sidx, zidx], axis=0)
    tv = jnp.concatenate([svals, jnp.zeros((_KSEL, 1), f32)], axis=0)

    sel17 = jnp.concatenate(
        [(tii == iota_l).astype(f32), jnp.ones((1, _T), f32)], axis=0)
    g17 = jnp.dot(sel17, xb, preferred_element_type=f32)
    xg = g17[0:_KEFF, :]
    xmean = g17[_KEFF:_KEFF + 1, :] * (1.0 / _T)

    i8f = i8r.astype(f32)
    cn = jnp.sum(p8r * (i8r <= tii).astype(f32), axis=1,
                 keepdims=True) / denom
    mean_cn = jnp.sum(p8r * (_T - i8f), keepdims=True) / (denom * _T)

    posn = tii.astype(f32) * (1.0 / _T)
    dvec = jnp.concatenate([xg, tv, posn, cn], axis=1)
    mp = jnp.full((1, 1), (_T - 1) / (2.0 * _T), f32)
    dmean = jnp.concatenate(
        [xmean, sump * (1.0 / _T), mp, mean_cn], axis=1)
    c = dvec - dmean
    c = c / (jnp.sqrt(jnp.sum(c * c, axis=1, keepdims=True)) + 1e-6)
    zz = (c - mu_ref[...]) / sig_ref[...]
    lif = jnp.tanh(jnp.dot(zz, wl_ref[...], preferred_element_type=f32)
                   + bl_ref[...])
    lif = lif / (jnp.sqrt(jnp.sum(lif * lif, axis=1, keepdims=True)) + 1e-6)
    tok_ref[0] = (jnp.dot(lif, wp_ref[...], preferred_element_type=f32)
                  + bp_ref[...])

    u8 = jnp.tanh(jnp.dot(xg[0:_KSEL, :], wu, preferred_element_type=f32)
                  + bur_ref[...])
    w8 = tv[0:_KSEL, :] / denom
    ctx = jnp.sum(w8 * u8, axis=0, keepdims=True)
    inp = jnp.concatenate([ctx, fb_ref[0]], axis=1)
    xh = jnp.concatenate([inp, m0r], axis=1)
    zg = jax.nn.sigmoid(jnp.dot(xh, wz_ref[...], preferred_element_type=f32)
                        + bz_ref[...])
    rg = jax.nn.sigmoid(jnp.dot(xh, wr_ref[...], preferred_element_type=f32)
                        + br_ref[...])
    xrh = jnp.concatenate([inp, rg * m0r], axis=1)
    hh = jnp.tanh(jnp.dot(xrh, wh_ref[...], preferred_element_type=f32)
                  + bh_ref[...])
    m1 = (1.0 - zg) * m0r + zg * hh
    mem_ref[0] = jnp.concatenate([m0r, m1], axis=0)


def kernel(x, feedback, params):
    p = params
    B, T, _ = x.shape
    f32 = jnp.float32

    full = lambda shape: pl.BlockSpec(shape, lambda b: tuple(0 for _ in shape))
    in_specs = [
        pl.BlockSpec((1, T, _DIN), lambda b: (b, 0, 0)),
        pl.BlockSpec((1, 1, 1), lambda b: (b, 0, 0)),
        full((_DIN, _DIN)), full((1, _DIN)), full((_DIN, 1)),
        full((_DIN, _DA)), full((_DA, 1)),
        full((_DA, _DM)), full((1, _DM)),
        full((_DA, 1)), full((1, 1)), full((1, T)),
        full((_DM * 2 + 1, _DM)), full((1, _DM)),
        full((_DM * 2 + 1, _DM)), full((1, _DM)),
        full((_DM * 2 + 1, _DM)), full((1, _DM)),
        full((1, _DIN + 3)), full((1, _DIN + 3)),
        full((_DIN + 3, _KLAT)), full((1, _KLAT)),
        full((_KLAT, _DMODEL)), full((1, _DMODEL)),
    ]
    out_specs = (
        pl.BlockSpec((1, 1, T), lambda b: (b, 0, 0)),
        pl.BlockSpec((1, _KEFF, _DMODEL), lambda b: (b, 0, 0)),
        pl.BlockSpec((1, 2, _DM), lambda b: (b, 0, 0)),
    )
    all_y, tokens, mem = pl.pallas_call(
        _body,
        grid=(B,),
        in_specs=in_specs,
        out_specs=out_specs,
        out_shape=(
            jax.ShapeDtypeStruct((B, 1, T), f32),
            jax.ShapeDtypeStruct((B, _KEFF, _DMODEL), f32),
            jax.ShapeDtypeStruct((B, 2, _DM), f32),
        ),
    )(
        x, feedback.reshape(B, 1, 1),
        p['W_u'], p['b_u'].reshape(1, -1), p['b_u'].reshape(-1, 1),
        p['W_a'], p['b_a'].reshape(-1, 1),
        p['W_ma'].T, p['m0'].reshape(1, -1),
        p['w_s'].reshape(-1, 1), p['b_s'].reshape(1, 1),
        p['pos_bias'][:T].reshape(1, -1),
        p['W_z'], p['b_z'].reshape(1, -1),
        p['W_r'], p['b_r'].reshape(1, -1),
        p['W_h'], p['b_h'].reshape(1, -1),
        p['mu'].reshape(1, -1), p['sigma'].reshape(1, -1),
        p['W_lift'], p['b_lift'].reshape(1, -1),
        p['W_proj'], p['b_proj'].reshape(1, -1),
    )
    y_star = all_y[:, 0, :]
    return tokens, y_star, all_y, mem
